# baseline (speedup 1.0000x reference)
import functools

import jax
import jax.numpy as jnp
from jax import lax
from jax.experimental import pallas as pl
from jax.experimental.pallas import tpu as pltpu
from jax.experimental.pallas import tpu_sc as plsc

N = 10000
D = 128
NP = 10240
NW = 32
CHUNK = 128
RPT = NP // 16

_mesh = plsc.VectorSubcoreMesh(core_axis_name="c", subcore_axis_name="s")


def _deg_body(ei_hbm, zer_hbm, out_hbm, dst_v, ones_v, dacc, sem):
    ncht = ei_hbm.shape[1]
    nb = ncht // NW
    c = lax.axis_index("c")
    s = lax.axis_index("s")
    wid = c * 16 + s

    @pl.loop(0, CHUNK // 16)
    def _(i):
        ones_v[pl.ds(i * 16, 16)] = jnp.ones((16,), jnp.float32)

    pltpu.sync_copy(zer_hbm, dacc.at[pl.ds(s * RPT, RPT)])
    pltpu.sync_copy(ei_hbm.at[1, pl.ds(wid * nb, nb)], dst_v)
    plsc.subcore_barrier()

    @pl.loop(0, nb)
    def _(j):
        pltpu.async_copy(ones_v, dacc.at[dst_v.at[j]], sem, add=True)

    @pl.loop(0, nb)
    def _(j):
        pltpu.make_async_copy(ones_v, dacc.at[dst_v.at[0]], sem).wait()

    plsc.subcore_barrier()
    sl = pl.ds(s * RPT, RPT)
    pltpu.sync_copy(dacc.at[sl], out_hbm.at[c, sl])


def _deg_call(ei3, zer1):
    ncht = ei3.shape[1]
    f = functools.partial(
        pl.kernel,
        out_type=jax.ShapeDtypeStruct((2, NP), jnp.float32),
        mesh=_mesh,
        scratch_types=[
            pltpu.VMEM((ncht // NW, CHUNK), jnp.int32),
            pltpu.VMEM((CHUNK,), jnp.float32),
            pltpu.VMEM_SHARED((NP,), jnp.float32),
            pltpu.SemaphoreType.DMA,
        ],
    )(_deg_body)
    return f(ei3, zer1)


def _edge_body(g_hbm, eif_hbm, zer_hbm, out_hbm, ring, buf_a, buf_b, acc,
               sem_i, sem_a, sem_b):
    ehalf = eif_hbm.shape[0] // 2
    ncht = ehalf // CHUNK
    nk = ncht // NW
    c = lax.axis_index("c")
    s = lax.axis_index("s")
    wid = c * 16 + s

    def idx_fetch(slot, k):
        for kk in (0, 1):
            off = (wid + (k + kk) * NW) * CHUNK
            pltpu.async_copy(eif_hbm.at[pl.ds(off, CHUNK)], ring.at[slot, kk, 0], sem_i)
            pltpu.async_copy(eif_hbm.at[pl.ds(ehalf + off, CHUNK)], ring.at[slot, kk, 1], sem_i)

    def wait_idx():
        for _ in range(4):
            pltpu.make_async_copy(eif_hbm.at[pl.ds(0, CHUNK)], ring.at[0, 0, 0], sem_i).wait()

    def gather(slot, k, buf, sem):
        pltpu.async_copy(g_hbm.at[ring.at[slot, k, 0]], buf, sem)

    def wait_g(buf, sem):
        pltpu.make_async_copy(g_hbm.at[ring.at[0, 0, 0]], buf, sem).wait()

    def scat(slot, k, buf):
        pltpu.sync_copy(buf, acc.at[ring.at[slot, k, 1]], add=True)

    idx_fetch(0, 0)
    pltpu.sync_copy(zer_hbm, acc.at[pl.ds(s * RPT, RPT)])
    wait_idx()
    idx_fetch(1, 2)
    gather(0, 0, buf_a, sem_a)
    gather(0, 1, buf_b, sem_b)
    plsc.subcore_barrier()

    @pl.loop(0, nk - 2, step=2)
    def _(j):
        p = (j // 2) % 2
        q = 1 - p
        wait_g(buf_a, sem_a)
        scat(p, 0, buf_a)
        wait_idx()
        gather(q, 0, buf_a, sem_a)
        wait_g(buf_b, sem_b)
        scat(p, 1, buf_b)
        gather(q, 1, buf_b, sem_b)
        idx_fetch(p, j + 4)

    pe = ((nk - 2) // 2) % 2
    wait_g(buf_a, sem_a)
    scat(pe, 0, buf_a)
    wait_g(buf_b, sem_b)
    scat(pe, 1, buf_b)
    wait_idx()

    plsc.subcore_barrier()
    sl = pl.ds(s * RPT, RPT)
    pltpu.sync_copy(acc.at[sl], out_hbm.at[c, sl])


def _edge_call(g, eif, zer2):
    f = functools.partial(
        pl.kernel,
        out_type=jax.ShapeDtypeStruct((2, NP, D), jnp.float32),
        mesh=_mesh,
        scratch_types=[
            pltpu.VMEM((2, 2, 2, CHUNK), jnp.int32),
            pltpu.VMEM((CHUNK, D), jnp.float32),
            pltpu.VMEM((CHUNK, D), jnp.float32),
            pltpu.VMEM_SHARED((NP, D), jnp.float32),
            pltpu.SemaphoreType.DMA,
            pltpu.SemaphoreType.DMA,
            pltpu.SemaphoreType.DMA,
        ],
    )(_edge_body)
    return f(g, eif, zer2)


def _tc1_body(x_ref, w_ref, degp_ref, o_ref):
    dinv = lax.rsqrt(degp_ref[0] + degp_ref[1] + 1.0)
    h = jnp.dot(x_ref[...], w_ref[...], preferred_element_type=jnp.float32,
                precision=lax.Precision.HIGHEST)
    o_ref[...] = h * dinv


def _tc2_body(s_ref, g_ref, degp_ref, w_ref, b_ref, o_ref):
    dinv = lax.rsqrt(degp_ref[0] + degp_ref[1] + 1.0)
    u = jnp.maximum(dinv * (s_ref[0] + s_ref[1] + g_ref[...]) + b_ref[...], 0.0)
    h = jnp.dot(u, w_ref[...], preferred_element_type=jnp.float32,
                precision=lax.Precision.HIGHEST)
    o_ref[...] = h * dinv


def _tc3_body(s_ref, g_ref, degp_ref, b_ref, o_ref):
    dinv = lax.rsqrt(degp_ref[0] + degp_ref[1] + 1.0)
    o_ref[...] = dinv * (s_ref[0] + s_ref[1] + g_ref[...]) + b_ref[...]


_out_np = jax.ShapeDtypeStruct((NP, D), jnp.float32)
_tc1 = pl.pallas_call(_tc1_body, out_shape=_out_np)
_tc2 = pl.pallas_call(_tc2_body, out_shape=_out_np)
_tc3 = pl.pallas_call(_tc3_body, out_shape=_out_np)


def kernel(x, edge_index, cache_name, W1, b1, W2, b2):
    e = edge_index.shape[1]
    blk = 2 * NW * CHUNK
    ep = ((e + blk - 1) // blk) * blk
    padv = jnp.broadcast_to(N + (jnp.arange(ep - e, dtype=jnp.int32) % 32),
                            (2, ep - e))
    eip = jnp.concatenate([edge_index, padv], axis=1)
    ei3 = eip.reshape(2, ep // CHUNK, CHUNK)
    eif = eip.reshape(2 * ep)
    xp = jnp.pad(x, ((0, NP - N), (0, 0)))
    zer1 = jnp.zeros((RPT,), jnp.float32)
    zer2 = jnp.zeros((RPT, D), jnp.float32)

    degp = _deg_call(ei3, zer1)[:, :, None]
    g1 = _tc1(xp, W1, degp)
    s1 = _edge_call(g1, eif, zer2)
    g2 = _tc2(s1, g1, degp, W2, b1.reshape(1, D))
    s2 = _edge_call(g2, eif, zer2)
    out = _tc3(s2, g2, degp, b2.reshape(1, D))
    return out[:N]


# speedup vs baseline: 1.0917x; 1.0917x over previous
import functools

import jax
import jax.numpy as jnp
from jax import lax
from jax.experimental import pallas as pl
from jax.experimental.pallas import tpu as pltpu
from jax.experimental.pallas import tpu_sc as plsc

N = 10000
D = 128
NP = 10240
NW = 32
CHUNK = 128
RPT = NP // 16

_mesh = plsc.VectorSubcoreMesh(core_axis_name="c", subcore_axis_name="s")


def _deg_body(ei_hbm, zer_hbm, out_hbm, dst_v, ones_v, dacc, sem):
    ncht = ei_hbm.shape[1]
    nb = ncht // NW
    c = lax.axis_index("c")
    s = lax.axis_index("s")
    wid = c * 16 + s

    @pl.loop(0, CHUNK // 16)
    def _(i):
        ones_v[pl.ds(i * 16, 16)] = jnp.ones((16,), jnp.float32)

    pltpu.sync_copy(zer_hbm, dacc.at[pl.ds(s * RPT, RPT)])
    pltpu.sync_copy(ei_hbm.at[1, pl.ds(wid * nb, nb)], dst_v)
    plsc.subcore_barrier()

    @pl.loop(0, nb)
    def _(j):
        pltpu.async_copy(ones_v, dacc.at[dst_v.at[j]], sem, add=True)

    @pl.loop(0, nb)
    def _(j):
        pltpu.make_async_copy(ones_v, dacc.at[dst_v.at[0]], sem).wait()

    plsc.subcore_barrier()
    sl = pl.ds(s * RPT, RPT)
    pltpu.sync_copy(dacc.at[sl], out_hbm.at[c, sl])


def _deg_call(ei3, zer1):
    ncht = ei3.shape[1]
    f = functools.partial(
        pl.kernel,
        out_type=jax.ShapeDtypeStruct((2, NP), jnp.float32),
        mesh=_mesh,
        scratch_types=[
            pltpu.VMEM((ncht // NW, CHUNK), jnp.int32),
            pltpu.VMEM((CHUNK,), jnp.float32),
            pltpu.VMEM_SHARED((NP,), jnp.float32),
            pltpu.SemaphoreType.DMA,
        ],
    )(_deg_body)
    return f(ei3, zer1)


def _edge_body(g_hbm, ei_hbm, zer_hbm, out_hbm, ring, buf_a, buf_b, acc,
               sem_i, sem_a, sem_b):
    ncht = ei_hbm.shape[1]
    nk = ncht // NW
    nblk = nk // 8
    c = lax.axis_index("c")
    s = lax.axis_index("s")
    wid = c * 16 + s
    base = wid * nk

    def idx_fetch(slot, m):
        off = jnp.minimum(base + 8 * m, ncht - 8)
        pltpu.async_copy(ei_hbm.at[0, pl.ds(off, 8)], ring.at[slot, 0], sem_i)
        pltpu.async_copy(ei_hbm.at[1, pl.ds(off, 8)], ring.at[slot, 1], sem_i)

    def wait_idx():
        for _ in range(2):
            pltpu.make_async_copy(ei_hbm.at[0, pl.ds(0, 8)], ring.at[0, 0],
                                  sem_i).wait()

    def gather(slot, r, buf, sem):
        pltpu.async_copy(g_hbm.at[ring.at[slot, 0, r]], buf, sem)

    def wait_g(buf, sem):
        pltpu.make_async_copy(g_hbm.at[ring.at[0, 0, 0]], buf, sem).wait()

    def scat(slot, r, buf):
        pltpu.sync_copy(buf, acc.at[ring.at[slot, 1, r]], add=True)

    idx_fetch(0, 0)
    pltpu.sync_copy(zer_hbm, acc.at[pl.ds(s * RPT, RPT)])
    wait_idx()
    idx_fetch(1, 1)
    gather(0, 0, buf_a, sem_a)
    gather(0, 1, buf_b, sem_b)
    plsc.subcore_barrier()

    @pl.loop(0, nblk - 1)
    def _(m):
        p = m % 2
        q = 1 - p
        for t in range(4):
            ra, rb = 2 * t, 2 * t + 1
            wait_g(buf_a, sem_a)
            scat(p, ra, buf_a)
            if t < 3:
                gather(p, ra + 2, buf_a, sem_a)
                wait_g(buf_b, sem_b)
                scat(p, rb, buf_b)
                gather(p, rb + 2, buf_b, sem_b)
            else:
                wait_idx()
                gather(q, 0, buf_a, sem_a)
                wait_g(buf_b, sem_b)
                scat(p, rb, buf_b)
                gather(q, 1, buf_b, sem_b)
                idx_fetch(p, m + 2)

    pe = (nblk - 1) % 2
    for t in range(4):
        ra, rb = 2 * t, 2 * t + 1
        wait_g(buf_a, sem_a)
        scat(pe, ra, buf_a)
        if t < 3:
            gather(pe, ra + 2, buf_a, sem_a)
        wait_g(buf_b, sem_b)
        scat(pe, rb, buf_b)
        if t < 3:
            gather(pe, rb + 2, buf_b, sem_b)
    wait_idx()

    plsc.subcore_barrier()
    sl = pl.ds(s * RPT, RPT)
    pltpu.sync_copy(acc.at[sl], out_hbm.at[c, sl])


def _edge_call(g, ei3, zer2):
    f = functools.partial(
        pl.kernel,
        out_type=jax.ShapeDtypeStruct((2, NP, D), jnp.float32),
        mesh=_mesh,
        scratch_types=[
            pltpu.VMEM((2, 2, 8, CHUNK), jnp.int32),
            pltpu.VMEM((CHUNK, D), jnp.float32),
            pltpu.VMEM((CHUNK, D), jnp.float32),
            pltpu.VMEM_SHARED((NP, D), jnp.float32),
            pltpu.SemaphoreType.DMA,
            pltpu.SemaphoreType.DMA,
            pltpu.SemaphoreType.DMA,
        ],
    )(_edge_body)
    return f(g, ei3, zer2)


def _tc1_body(x_ref, w_ref, degp_ref, o_ref):
    dinv = lax.rsqrt(degp_ref[0] + degp_ref[1] + 1.0)
    h = jnp.dot(x_ref[...], w_ref[...], preferred_element_type=jnp.float32,
                precision=lax.Precision.HIGHEST)
    o_ref[...] = h * dinv


def _tc2_body(s_ref, g_ref, degp_ref, w_ref, b_ref, o_ref):
    dinv = lax.rsqrt(degp_ref[0] + degp_ref[1] + 1.0)
    u = jnp.maximum(dinv * (s_ref[0] + s_ref[1] + g_ref[...]) + b_ref[...], 0.0)
    h = jnp.dot(u, w_ref[...], preferred_element_type=jnp.float32,
                precision=lax.Precision.HIGHEST)
    o_ref[...] = h * dinv


def _tc3_body(s_ref, g_ref, degp_ref, b_ref, o_ref):
    dinv = lax.rsqrt(degp_ref[0] + degp_ref[1] + 1.0)
    o_ref[...] = dinv * (s_ref[0] + s_ref[1] + g_ref[...]) + b_ref[...]


_B1 = 1280
_B3 = 2000

_row = lambda i: (i, 0)
_full = lambda i: (0, 0)
_mid = lambda i: (0, i, 0)

_tc1 = pl.pallas_call(
    _tc1_body,
    grid=(NP // _B1,),
    in_specs=[pl.BlockSpec((_B1, D), _row),
              pl.BlockSpec((D, D), _full),
              pl.BlockSpec((2, _B1, 1), _mid)],
    out_specs=pl.BlockSpec((_B1, D), _row),
    out_shape=jax.ShapeDtypeStruct((NP, D), jnp.float32),
)
_tc2 = pl.pallas_call(
    _tc2_body,
    grid=(NP // _B1,),
    in_specs=[pl.BlockSpec((2, _B1, D), _mid),
              pl.BlockSpec((_B1, D), _row),
              pl.BlockSpec((2, _B1, 1), _mid),
              pl.BlockSpec((D, D), _full),
              pl.BlockSpec((1, D), _full)],
    out_specs=pl.BlockSpec((_B1, D), _row),
    out_shape=jax.ShapeDtypeStruct((NP, D), jnp.float32),
)
_tc3 = pl.pallas_call(
    _tc3_body,
    grid=(N // _B3,),
    in_specs=[pl.BlockSpec((2, _B3, D), _mid),
              pl.BlockSpec((_B3, D), _row),
              pl.BlockSpec((2, _B3, 1), _mid),
              pl.BlockSpec((1, D), _full)],
    out_specs=pl.BlockSpec((_B3, D), _row),
    out_shape=jax.ShapeDtypeStruct((N, D), jnp.float32),
)


def kernel(x, edge_index, cache_name, W1, b1, W2, b2):
    e = edge_index.shape[1]
    if e % CHUNK:
        pe_ = CHUNK - e % CHUNK
        padv = jnp.broadcast_to(N + (jnp.arange(pe_, dtype=jnp.int32) % 32),
                                (2, pe_))
        edge_index = jnp.concatenate([edge_index, padv], axis=1)
        e += pe_
    nch0 = e // CHUNK
    ncht = ((nch0 + 255) // 256) * 256
    ei3r = edge_index.reshape(2, nch0, CHUNK)
    padc = jnp.broadcast_to(
        N + (jnp.arange((ncht - nch0) * CHUNK, dtype=jnp.int32) % 32)
        .reshape(1, ncht - nch0, CHUNK), (2, ncht - nch0, CHUNK))
    ei3 = jnp.concatenate([ei3r, padc], axis=1)
    xp = jnp.pad(x, ((0, NP - N), (0, 0)))
    zer1 = jnp.zeros((RPT,), jnp.float32)
    zer2 = jnp.zeros((RPT, D), jnp.float32)

    degp = _deg_call(ei3, zer1)[:, :, None]
    g1 = _tc1(xp, W1, degp)
    s1 = _edge_call(g1, ei3, zer2)
    g2 = _tc2(s1, g1, degp, W2, b1.reshape(1, D))
    s2 = _edge_call(g2, ei3, zer2)
    return _tc3(s2, g2, degp, b2.reshape(1, D))


# speedup vs baseline: 1.1259x; 1.0314x over previous
import functools

import jax
import jax.numpy as jnp
from jax import lax
from jax.experimental import pallas as pl
from jax.experimental.pallas import tpu as pltpu
from jax.experimental.pallas import tpu_sc as plsc

N = 10000
D = 128
NP = 10240
NW = 32
CHUNK = 128
RPT = NP // 16

_mesh = plsc.VectorSubcoreMesh(core_axis_name="c", subcore_axis_name="s")


def _deg_body(ei_hbm, zer_hbm, out_hbm, dst_v, ones_v, dacc, sem):
    ncht = ei_hbm.shape[1]
    nb = ncht // NW
    c = lax.axis_index("c")
    s = lax.axis_index("s")
    wid = c * 16 + s

    @pl.loop(0, CHUNK // 16)
    def _(i):
        ones_v[pl.ds(i * 16, 16)] = jnp.ones((16,), jnp.float32)

    pltpu.sync_copy(zer_hbm, dacc.at[pl.ds(s * RPT, RPT)])
    pltpu.sync_copy(ei_hbm.at[1, pl.ds(wid * nb, nb)], dst_v)
    plsc.subcore_barrier()

    @pl.loop(0, nb)
    def _(j):
        pltpu.async_copy(ones_v, dacc.at[dst_v.at[j]], sem, add=True)

    @pl.loop(0, nb)
    def _(j):
        pltpu.make_async_copy(ones_v, dacc.at[dst_v.at[0]], sem).wait()

    plsc.subcore_barrier()
    sl = pl.ds(s * RPT, RPT)
    pltpu.sync_copy(dacc.at[sl], out_hbm.at[c, sl])


def _deg_call(ei3, zer1):
    ncht = ei3.shape[1]
    f = functools.partial(
        pl.kernel,
        out_type=jax.ShapeDtypeStruct((2, NP), jnp.float32),
        mesh=_mesh,
        scratch_types=[
            pltpu.VMEM((ncht // NW, CHUNK), jnp.int32),
            pltpu.VMEM((CHUNK,), jnp.float32),
            pltpu.VMEM_SHARED((NP,), jnp.float32),
            pltpu.SemaphoreType.DMA,
        ],
    )(_deg_body)
    return f(ei3, zer1)


def _edge_body(g_hbm, ei_hbm, zer_hbm, out_hbm, ring, buf_a, buf_b, acc,
               sem_i, sem_a, sem_b):
    ncht = ei_hbm.shape[1]
    nk = ncht // NW
    nblk = nk // 8
    c = lax.axis_index("c")
    s = lax.axis_index("s")
    wid = c * 16 + s
    base = wid * nk

    def idx_fetch(slot, m):
        off = jnp.minimum(base + 8 * m, ncht - 8)
        pltpu.async_copy(ei_hbm.at[0, pl.ds(off, 8)], ring.at[slot, 0], sem_i)
        pltpu.async_copy(ei_hbm.at[1, pl.ds(off, 8)], ring.at[slot, 1], sem_i)

    def wait_idx():
        for _ in range(2):
            pltpu.make_async_copy(ei_hbm.at[0, pl.ds(0, 8)], ring.at[0, 0],
                                  sem_i).wait()

    def gather(slot, r, buf, sem):
        pltpu.async_copy(g_hbm.at[ring.at[slot, 0, r]], buf, sem)

    def wait_g(buf, sem):
        pltpu.make_async_copy(g_hbm.at[ring.at[0, 0, 0]], buf, sem).wait()

    def scat(slot, r, buf):
        pltpu.sync_copy(buf, acc.at[ring.at[slot, 1, r]], add=True)

    idx_fetch(0, 0)
    pltpu.sync_copy(zer_hbm, acc.at[pl.ds(s * RPT, RPT)])
    wait_idx()
    idx_fetch(1, 1)
    gather(0, 0, buf_a, sem_a)
    gather(0, 1, buf_b, sem_b)
    plsc.subcore_barrier()

    @pl.loop(0, nblk - 1)
    def _(m):
        p = m % 2
        q = 1 - p
        for t in range(4):
            ra, rb = 2 * t, 2 * t + 1
            wait_g(buf_a, sem_a)
            scat(p, ra, buf_a)
            if t < 3:
                gather(p, ra + 2, buf_a, sem_a)
                wait_g(buf_b, sem_b)
                scat(p, rb, buf_b)
                gather(p, rb + 2, buf_b, sem_b)
            else:
                wait_idx()
                gather(q, 0, buf_a, sem_a)
                wait_g(buf_b, sem_b)
                scat(p, rb, buf_b)
                gather(q, 1, buf_b, sem_b)
                idx_fetch(p, m + 2)

    pe = (nblk - 1) % 2
    for t in range(4):
        ra, rb = 2 * t, 2 * t + 1
        wait_g(buf_a, sem_a)
        scat(pe, ra, buf_a)
        if t < 3:
            gather(pe, ra + 2, buf_a, sem_a)
        wait_g(buf_b, sem_b)
        scat(pe, rb, buf_b)
        if t < 3:
            gather(pe, rb + 2, buf_b, sem_b)
    wait_idx()

    plsc.subcore_barrier()
    sl = pl.ds(s * RPT, RPT)
    pltpu.sync_copy(acc.at[sl], out_hbm.at[c, sl])


def _edge_call(g, ei3, zer2):
    f = functools.partial(
        pl.kernel,
        out_type=jax.ShapeDtypeStruct((2, NP, D), jnp.float32),
        mesh=_mesh,
        scratch_types=[
            pltpu.VMEM((2, 2, 8, CHUNK), jnp.int32),
            pltpu.VMEM((CHUNK, D), jnp.float32),
            pltpu.VMEM((CHUNK, D), jnp.float32),
            pltpu.VMEM_SHARED((NP, D), jnp.float32),
            pltpu.SemaphoreType.DMA,
            pltpu.SemaphoreType.DMA,
            pltpu.SemaphoreType.DMA,
        ],
    )(_edge_body)
    return f(g, ei3, zer2)


def _dinv_col(degp_ref):
    deg = degp_ref[0:1, :] + degp_ref[1:2, :] + 1.0
    return jnp.transpose(lax.rsqrt(deg), (1, 0))


def _tc1_body(x_ref, w_ref, degp_ref, o_ref):
    h = jnp.dot(x_ref[...], w_ref[...], preferred_element_type=jnp.float32,
                precision=lax.Precision.HIGHEST)
    o_ref[...] = h * _dinv_col(degp_ref)


def _tc2_body(s_ref, g_ref, degp_ref, w_ref, b_ref, o_ref):
    dinv = _dinv_col(degp_ref)
    u = jnp.maximum(dinv * (s_ref[0] + s_ref[1] + g_ref[...]) + b_ref[...], 0.0)
    h = jnp.dot(u, w_ref[...], preferred_element_type=jnp.float32,
                precision=lax.Precision.HIGHEST)
    o_ref[...] = h * dinv


def _tc3_body(s_ref, g_ref, degp_ref, b_ref, o_ref):
    o_ref[...] = (_dinv_col(degp_ref) * (s_ref[0] + s_ref[1] + g_ref[...])
                  + b_ref[...])


_B1 = 1280

_row = lambda i: (i, 0)
_full = lambda i: (0, 0)
_mid = lambda i: (0, i, 0)
_rowT = lambda i: (0, i)

_tc1 = pl.pallas_call(
    _tc1_body,
    grid=(NP // _B1,),
    in_specs=[pl.BlockSpec((_B1, D), _row),
              pl.BlockSpec((D, D), _full),
              pl.BlockSpec((2, _B1), _rowT)],
    out_specs=pl.BlockSpec((_B1, D), _row),
    out_shape=jax.ShapeDtypeStruct((NP, D), jnp.float32),
)
_tc2 = pl.pallas_call(
    _tc2_body,
    grid=(NP // _B1,),
    in_specs=[pl.BlockSpec((2, _B1, D), _mid),
              pl.BlockSpec((_B1, D), _row),
              pl.BlockSpec((2, _B1), _rowT),
              pl.BlockSpec((D, D), _full),
              pl.BlockSpec((1, D), _full)],
    out_specs=pl.BlockSpec((_B1, D), _row),
    out_shape=jax.ShapeDtypeStruct((NP, D), jnp.float32),
)
_tc3 = pl.pallas_call(
    _tc3_body,
    grid=(NP // _B1,),
    in_specs=[pl.BlockSpec((2, _B1, D), _mid),
              pl.BlockSpec((_B1, D), _row),
              pl.BlockSpec((2, _B1), _rowT),
              pl.BlockSpec((1, D), _full)],
    out_specs=pl.BlockSpec((_B1, D), _row),
    out_shape=jax.ShapeDtypeStruct((N, D), jnp.float32),
)


def kernel(x, edge_index, cache_name, W1, b1, W2, b2):
    e = edge_index.shape[1]
    if e % CHUNK:
        pe_ = CHUNK - e % CHUNK
        padv = jnp.broadcast_to(N + (jnp.arange(pe_, dtype=jnp.int32) % 32),
                                (2, pe_))
        edge_index = jnp.concatenate([edge_index, padv], axis=1)
        e += pe_
    nch0 = e // CHUNK
    ncht = ((nch0 + 255) // 256) * 256
    ei3r = edge_index.reshape(2, nch0, CHUNK)
    padc = jnp.broadcast_to(
        N + (jnp.arange((ncht - nch0) * CHUNK, dtype=jnp.int32) % 32)
        .reshape(1, ncht - nch0, CHUNK), (2, ncht - nch0, CHUNK))
    ei3 = jnp.concatenate([ei3r, padc], axis=1)
    xp = jnp.pad(x, ((0, NP - N), (0, 0)))
    zer1 = jnp.zeros((RPT,), jnp.float32)
    zer2 = jnp.zeros((RPT, D), jnp.float32)

    degp = _deg_call(ei3, zer1)
    g1 = _tc1(xp, W1, degp)
    s1 = _edge_call(g1, ei3, zer2)
    g2 = _tc2(s1, g1, degp, W2, b1.reshape(1, D))
    s2 = _edge_call(g2, ei3, zer2)
    return _tc3(s2, g2, degp, b2.reshape(1, D))
